# baseline (speedup 1.0000x reference)
import functools

import jax
import jax.numpy as jnp
from jax import lax
from jax.experimental import pallas as pl
from jax.experimental.pallas import tpu as pltpu
from jax.experimental.pallas import tpu_sc as plsc

B = 16384
V = 213
D_OUT = 10
NW = 32
CPW = B // NW
CB = 64
NB = CPW // CB
SUB = 256
L = 16


def _sc_body(xt_hbm, w_hbm, b_hbm, out_hbm, xbuf, wbuf, obuf, bbuf):
    cid = lax.axis_index("c")
    sid = lax.axis_index("s")
    wid = sid * 2 + cid
    base = wid * CPW
    pltpu.sync_copy(w_hbm, wbuf)
    pltpu.sync_copy(b_hbm, bbuf)

    for half in range(2):
        h0 = half * SUB
        pltpu.sync_copy(xt_hbm.at[:, pl.ds(base + h0, SUB)], xbuf)
        for cb in range(SUB // CB):
            c0 = cb * CB

            def vstep(v, accs):
                xvs = [xbuf[v, pl.ds(c0 + j * L, L)] for j in range(4)]
                new = []
                for d in range(D_OUT):
                    wsp = wbuf[d, pl.ds(v * L, L)]
                    for j in range(4):
                        new.append(accs[d * 4 + j] + xvs[j] * wsp)
                return tuple(new)

            init = []
            for d in range(D_OUT):
                bsp = bbuf[d, :]
                for j in range(4):
                    init.append(bsp)
            accs = lax.fori_loop(0, V, vstep, tuple(init))
            for d in range(D_OUT):
                for j in range(4):
                    obuf[d, pl.ds(h0 + c0 + j * L, L)] = accs[d * 4 + j]

    pltpu.sync_copy(obuf, out_hbm.at[:, pl.ds(base, CPW)])


def kernel(x, W, b):
    xt = x.T
    mesh = plsc.VectorSubcoreMesh(core_axis_name="c", subcore_axis_name="s")
    sc = functools.partial(
        pl.kernel,
        mesh=mesh,
        out_type=jax.ShapeDtypeStruct((D_OUT, B), jnp.float32),
        scratch_types=[
            pltpu.VMEM((V, SUB), jnp.float32),
            pltpu.VMEM((D_OUT, V * L), jnp.float32),
            pltpu.VMEM((D_OUT, CPW), jnp.float32),
            pltpu.VMEM((D_OUT, L), jnp.float32),
        ],
    )(_sc_body)
    wsplat = jnp.broadcast_to(W[:, :, None], (D_OUT, V, L)).reshape(D_OUT, V * L)
    b2 = jnp.broadcast_to(b[:, None], (D_OUT, L))
    outT = sc(xt, wsplat, b2)
    return outT.T


# speedup vs baseline: 7.3995x; 7.3995x over previous
import jax
import jax.numpy as jnp
from jax.experimental import pallas as pl
from jax.experimental.pallas import tpu as pltpu

B = 16384
V = 213
D_OUT = 10
NCHUNK = 4
CH = B // NCHUNK


def _body(xt_hbm, w_ref, b_ref, out_hbm, xbuf, obuf, insem, outsem):
    in_cps = []
    for k in range(NCHUNK):
        cp = pltpu.make_async_copy(
            xt_hbm.at[:, pl.ds(k * CH, CH)], xbuf.at[k], insem.at[k]
        )
        cp.start(priority=0)
        in_cps.append(cp)
    out_cps = []
    for k in range(NCHUNK):
        in_cps[k].wait()
        obuf[:, pl.ds(k * CH, CH)] = (
            jnp.dot(w_ref[...], xbuf[k], preferred_element_type=jnp.float32)
            + b_ref[...]
        )
        ocp = pltpu.make_async_copy(
            obuf.at[:, pl.ds(k * CH, CH)],
            out_hbm.at[:, pl.ds(k * CH, CH)],
            outsem.at[k],
        )
        ocp.start(priority=1)
        out_cps.append(ocp)
    for k in range(NCHUNK):
        out_cps[k].wait()


def kernel(x, W, b):
    xt = x.T
    b2 = b.reshape(D_OUT, 1)
    outT = pl.pallas_call(
        _body,
        in_specs=[
            pl.BlockSpec(memory_space=pl.ANY),
            pl.BlockSpec((D_OUT, V), lambda: (0, 0)),
            pl.BlockSpec((D_OUT, 1), lambda: (0, 0)),
        ],
        out_specs=pl.BlockSpec(memory_space=pl.ANY),
        out_shape=jax.ShapeDtypeStruct((D_OUT, B), jnp.float32),
        scratch_shapes=[
            pltpu.VMEM((NCHUNK, V, CH), jnp.float32),
            pltpu.VMEM((D_OUT, B), jnp.float32),
            pltpu.SemaphoreType.DMA((NCHUNK,)),
            pltpu.SemaphoreType.DMA((NCHUNK,)),
        ],
    )(xt, W, b2)
    return outT.T
